# baseline (speedup 1.0000x reference)
import functools

import jax
import jax.numpy as jnp
from jax import lax
from jax.experimental import pallas as pl
from jax.experimental.pallas import tpu as pltpu
from jax.experimental.pallas import tpu_sc as plsc

NC = 2
NS = 16
NW = NC * NS
CH = 128
W = 8
LANES = 16
NCH0 = 128
NCH1 = 32


def _sc_degree(dst_w, n_acc, nch, stripe):
    mesh = plsc.VectorSubcoreMesh(core_axis_name="c", subcore_axis_name="s")

    @functools.partial(
        pl.kernel,
        out_type=jax.ShapeDtypeStruct((NC, n_acc), jnp.float32),
        mesh=mesh,
        scratch_types=[
            pltpu.VMEM((nch, CH), jnp.int32),
            pltpu.VMEM((CH,), jnp.float32),
            pltpu.VMEM((stripe,), jnp.float32),
            pltpu.VMEM_SHARED((n_acc,), jnp.float32),
        ],
    )
    def k(dst_hbm, out_hbm, dst_v, ones_v, zero_v, acc):
        cid = lax.axis_index("c")
        sid = lax.axis_index("s")
        wid = sid * NC + cid

        @pl.loop(0, CH, step=LANES)
        def _(c):
            ones_v[pl.ds(c, LANES)] = jnp.ones((LANES,), jnp.float32)

        @pl.loop(0, stripe, step=LANES)
        def _(c):
            zero_v[pl.ds(c, LANES)] = jnp.zeros((LANES,), jnp.float32)

        pltpu.sync_copy(zero_v, acc.at[pl.ds(sid * stripe, stripe)])
        plsc.subcore_barrier()

        pltpu.sync_copy(dst_hbm.at[wid], dst_v)

        @pl.loop(0, nch)
        def _(j):
            pltpu.sync_copy(ones_v, acc.at[dst_v.at[j]], add=True)

        plsc.subcore_barrier()
        pltpu.sync_copy(acc.at[pl.ds(sid * stripe, stripe)],
                        out_hbm.at[cid, pl.ds(sid * stripe, stripe)])

    return k(dst_w)


def _sc_aggregate(h, src_w, dst_w, n_acc, stripe, d):
    mesh = plsc.VectorSubcoreMesh(core_axis_name="c", subcore_axis_name="s")

    @functools.partial(
        pl.kernel,
        out_type=jax.ShapeDtypeStruct((NC, n_acc, d), jnp.float32),
        mesh=mesh,
        scratch_types=[
            pltpu.VMEM((W, CH), jnp.int32),
            pltpu.VMEM((W, CH), jnp.int32),
            pltpu.VMEM((W, CH), jnp.int32),
            pltpu.VMEM((W, CH), jnp.int32),
            pltpu.VMEM((CH, d), jnp.float32),
            pltpu.VMEM((CH, d), jnp.float32),
            pltpu.VMEM_SHARED((n_acc, d), jnp.float32),
            pltpu.SemaphoreType.DMA,
            pltpu.SemaphoreType.DMA,
            pltpu.SemaphoreType.DMA,
            pltpu.SemaphoreType.DMA,
        ],
    )
    def k(h_hbm, src_hbm, dst_hbm, out_hbm, swin0, swin1, dwin0, dwin1,
          buf0, buf1, acc, sem0, sem1, semA, semB):
        cid = lax.axis_index("c")
        sid = lax.axis_index("s")
        wid = sid * NC + cid
        nch = jnp.where(cid == 0, NCH0, NCH1)
        nwin = nch // W

        zvec = jnp.zeros((LANES,), jnp.float32)

        @pl.loop(0, CH)
        def _(r):
            @pl.loop(0, d, step=LANES)
            def _(c):
                buf0[r, pl.ds(c, LANES)] = zvec

        @pl.loop(0, stripe, step=CH)
        def _(r0):
            pltpu.sync_copy(buf0, acc.at[pl.ds(sid * stripe + r0, CH)])

        plsc.subcore_barrier()

        def load_win(sw, dw, wi, sem):
            off = pl.multiple_of(wi * W, W)
            pltpu.make_async_copy(src_hbm.at[wid, pl.ds(off, W)], sw,
                                  sem).start()
            pltpu.make_async_copy(dst_hbm.at[wid, pl.ds(off, W)], dw,
                                  sem).start()

        def wait_win(sw, dw, sem):
            pltpu.make_async_copy(src_hbm.at[wid, pl.ds(0, W)], sw,
                                  sem).wait()
            pltpu.make_async_copy(dst_hbm.at[wid, pl.ds(0, W)], dw,
                                  sem).wait()

        pltpu.sync_copy(src_hbm.at[wid, pl.ds(0, W)], swin0)
        pltpu.sync_copy(dst_hbm.at[wid, pl.ds(0, W)], dwin0)
        load_win(swin1, dwin1, 1, semB)
        pltpu.make_async_copy(h_hbm.at[swin0.at[0]], buf0, sem0).start()
        pltpu.make_async_copy(h_hbm.at[swin0.at[1]], buf1, sem1).start()

        def process_window(sw, dw, swn, wi):
            for jj in range(0, W, 2):
                for (jo, buf, sem) in ((jj, buf0, sem0), (jj + 1, buf1, sem1)):
                    pltpu.make_async_copy(h_hbm.at[sw.at[jo]], buf, sem).wait()
                    pltpu.sync_copy(buf, acc.at[dw.at[jo]], add=True)
                    nj = jo + 2
                    nidx = sw.at[nj] if nj < W else swn.at[nj - W]

                    @pl.when(wi * W + nj < nch)
                    def _():
                        pltpu.make_async_copy(h_hbm.at[nidx], buf, sem).start()

        @pl.loop(0, nwin, step=2)
        def _(w):
            wait_win(swin1, dwin1, semB)
            process_window(swin0, dwin0, swin1, w)

            @pl.when(w + 2 < nwin)
            def _():
                load_win(swin0, dwin0, w + 2, semA)
                wait_win(swin0, dwin0, semA)

            process_window(swin1, dwin1, swin0, w + 1)

            @pl.when(w + 3 < nwin)
            def _():
                load_win(swin1, dwin1, w + 3, semB)

        plsc.subcore_barrier()
        pltpu.sync_copy(acc.at[pl.ds(sid * stripe, stripe)],
                        out_hbm.at[cid, pl.ds(sid * stripe, stripe)])

    return k(h, src_w, dst_w)


def _dot(a, b):
    return jnp.dot(a, b, precision=lax.Precision.HIGHEST,
                   preferred_element_type=jnp.float32)


def _tc_matmul(x, w, br):
    n, d = x.shape

    def body(x_r, w_r, o_r):
        o_r[...] = _dot(x_r[...], w_r[...])

    return pl.pallas_call(
        body,
        grid=(n // br,),
        in_specs=[pl.BlockSpec((br, d), lambda i: (i, 0)),
                  pl.BlockSpec((d, d), lambda i: (0, 0))],
        out_specs=pl.BlockSpec((br, d), lambda i: (i, 0)),
        out_shape=jax.ShapeDtypeStruct((n, d), jnp.float32),
    )(x, w)


def _tc_scale(deg_parts, h, br):
    n, d = h.shape

    def body(d_r, h_r, dis_o, hp_o):
        dis = lax.rsqrt(d_r[0] + d_r[1] + 1.0)
        dis_o[...] = dis
        hp_o[...] = h_r[...] * dis

    return pl.pallas_call(
        body,
        grid=(n // br,),
        in_specs=[pl.BlockSpec((2, br, 1), lambda i: (0, i, 0)),
                  pl.BlockSpec((br, d), lambda i: (i, 0))],
        out_specs=[pl.BlockSpec((br, 1), lambda i: (i, 0)),
                   pl.BlockSpec((br, d), lambda i: (i, 0))],
        out_shape=[jax.ShapeDtypeStruct((n, 1), jnp.float32),
                   jax.ShapeDtypeStruct((n, d), jnp.float32)],
    )(deg_parts, h)


def _tc_finish_mm(parts, hp, dis, b, w, br):
    n, d = hp.shape

    def body(p_r, hp_r, dis_r, b_r, w_r, z_o, hp2_o):
        dis = dis_r[...]
        z = jnp.maximum(dis * (p_r[0] + p_r[1] + hp_r[...]) + b_r[...], 0.0)
        z_o[...] = z
        hp2_o[...] = _dot(z, w_r[...]) * dis

    return pl.pallas_call(
        body,
        grid=(n // br,),
        in_specs=[pl.BlockSpec((2, br, d), lambda i: (0, i, 0)),
                  pl.BlockSpec((br, d), lambda i: (i, 0)),
                  pl.BlockSpec((br, 1), lambda i: (i, 0)),
                  pl.BlockSpec((1, d), lambda i: (0, 0)),
                  pl.BlockSpec((d, d), lambda i: (0, 0))],
        out_specs=[pl.BlockSpec((br, d), lambda i: (i, 0)),
                   pl.BlockSpec((br, d), lambda i: (i, 0))],
        out_shape=[jax.ShapeDtypeStruct((n, d), jnp.float32),
                   jax.ShapeDtypeStruct((n, d), jnp.float32)],
    )(parts, hp, dis, b, w)


def _tc_finish(parts, hp, dis, b, br):
    n, d = hp.shape

    def body(p_r, hp_r, dis_r, b_r, o_r):
        o_r[...] = dis_r[...] * (p_r[0] + p_r[1] + hp_r[...]) + b_r[...]

    return pl.pallas_call(
        body,
        grid=(n // br,),
        in_specs=[pl.BlockSpec((2, br, d), lambda i: (0, i, 0)),
                  pl.BlockSpec((br, d), lambda i: (i, 0)),
                  pl.BlockSpec((br, 1), lambda i: (i, 0)),
                  pl.BlockSpec((1, d), lambda i: (0, 0))],
        out_specs=pl.BlockSpec((br, d), lambda i: (i, 0)),
        out_shape=jax.ShapeDtypeStruct((n, d), jnp.float32),
    )(parts, hp, dis, b)


def kernel(x, edge_index, W1, b1, W2, b2):
    n, d = x.shape
    e = edge_index.shape[1]

    stripe = -(-(n + 1) // (NS * CH)) * CH
    n_acc = NS * stripe
    br = 1000

    src = edge_index[0].astype(jnp.int32)
    dst = edge_index[1].astype(jnp.int32)

    e0_cap = NS * NCH0 * CH
    e1_cap = NS * NCH1 * CH
    e0 = min(e, e0_cap * e // (e0_cap + e1_cap) // CH * CH)

    def side(idx_arr, fill, lo, hi, cap, nch_rows):
        part = idx_arr[lo:hi]
        part = jnp.concatenate(
            [part, jnp.full((cap - (hi - lo),), fill, jnp.int32)])
        part = part.reshape(NS, nch_rows, CH)
        if nch_rows < NCH0:
            part = jnp.concatenate(
                [part, jnp.full((NS, NCH0 - nch_rows, CH), fill, jnp.int32)],
                axis=1)
        return part

    def both_sides(idx_arr, fill):
        a = side(idx_arr, fill, 0, e0, e0_cap, NCH0)
        b = side(idx_arr, fill, e0, e, e1_cap, NCH1)
        return jnp.stack([a, b], axis=1).reshape(NW, NCH0, CH)

    src_w = both_sides(src, 0)
    dst_w = both_sides(dst, n)

    nch_deg = -(-e // (NW * CH))
    e_pad = NW * nch_deg * CH
    dst_deg = jnp.concatenate(
        [dst, jnp.full((e_pad - e,), n, jnp.int32)]).reshape(NW, nch_deg, CH)

    b1r = b1.reshape(1, d).astype(jnp.float32)
    b2r = b2.reshape(1, d).astype(jnp.float32)

    deg_parts = _sc_degree(dst_deg, n_acc, nch_deg, stripe)
    h1 = _tc_matmul(x, W1, br)

    dis, h1p = _tc_scale(deg_parts.reshape(NC, n_acc, 1), h1, br)

    p = _sc_aggregate(h1p, src_w, dst_w, n_acc, stripe, d)
    z1, h2p = _tc_finish_mm(p, h1p, dis, b1r, W2, br)

    q = _sc_aggregate(h2p, src_w, dst_w, n_acc, stripe, d)
    out2 = _tc_finish(q, h2p, dis, b2r, br)

    return (x, z1, out2)


# speedup vs baseline: 1.1694x; 1.1694x over previous
import functools

import jax
import jax.numpy as jnp
from jax import lax
from jax.experimental import pallas as pl
from jax.experimental.pallas import tpu as pltpu
from jax.experimental.pallas import tpu_sc as plsc

NC = 2
NS = 16
NW = NC * NS
CH = 128
W = 8
LANES = 16
NCH0 = 80
NCH1 = 80


def _sc_degree(dst_w, n_acc, nch, stripe):
    mesh = plsc.VectorSubcoreMesh(core_axis_name="c", subcore_axis_name="s")

    @functools.partial(
        pl.kernel,
        out_type=jax.ShapeDtypeStruct((NC, n_acc), jnp.float32),
        mesh=mesh,
        scratch_types=[
            pltpu.VMEM((nch, CH), jnp.int32),
            pltpu.VMEM((CH,), jnp.float32),
            pltpu.VMEM((stripe,), jnp.float32),
            pltpu.VMEM_SHARED((n_acc,), jnp.float32),
        ],
    )
    def k(dst_hbm, out_hbm, dst_v, ones_v, zero_v, acc):
        cid = lax.axis_index("c")
        sid = lax.axis_index("s")
        wid = sid * NC + cid

        @pl.loop(0, CH, step=LANES)
        def _(c):
            ones_v[pl.ds(c, LANES)] = jnp.ones((LANES,), jnp.float32)

        @pl.loop(0, stripe, step=LANES)
        def _(c):
            zero_v[pl.ds(c, LANES)] = jnp.zeros((LANES,), jnp.float32)

        pltpu.sync_copy(zero_v, acc.at[pl.ds(sid * stripe, stripe)])
        plsc.subcore_barrier()

        pltpu.sync_copy(dst_hbm.at[wid], dst_v)

        @pl.loop(0, nch)
        def _(j):
            pltpu.sync_copy(ones_v, acc.at[dst_v.at[j]], add=True)

        plsc.subcore_barrier()
        pltpu.sync_copy(acc.at[pl.ds(sid * stripe, stripe)],
                        out_hbm.at[cid, pl.ds(sid * stripe, stripe)])

    return k(dst_w)


def _sc_aggregate(h, src_w, dst_w, n_acc, stripe, d):
    mesh = plsc.VectorSubcoreMesh(core_axis_name="c", subcore_axis_name="s")

    @functools.partial(
        pl.kernel,
        out_type=jax.ShapeDtypeStruct((NC, n_acc, d), jnp.float32),
        mesh=mesh,
        scratch_types=[
            pltpu.VMEM((W, CH), jnp.int32),
            pltpu.VMEM((W, CH), jnp.int32),
            pltpu.VMEM((W, CH), jnp.int32),
            pltpu.VMEM((W, CH), jnp.int32),
            pltpu.VMEM((CH, d // 2), jnp.int32),
            pltpu.VMEM((CH, d // 2), jnp.int32),
            pltpu.VMEM((CH, d), jnp.float32),
            pltpu.VMEM_SHARED((n_acc, d), jnp.float32),
            pltpu.SemaphoreType.DMA,
            pltpu.SemaphoreType.DMA,
            pltpu.SemaphoreType.DMA,
            pltpu.SemaphoreType.DMA,
        ],
        compiler_params=pltpu.CompilerParams(use_tc_tiling_on_sc=False,
                                             needs_layout_passes=False),
    )
    def k(h_hbm, src_hbm, dst_hbm, out_hbm, swin0, swin1, dwin0, dwin1,
          buf0, buf1, fbuf, acc, sem0, sem1, semA, semB):
        cid = lax.axis_index("c")
        sid = lax.axis_index("s")
        wid = sid * NC + cid
        nch = jnp.where(cid == 0, NCH0, NCH1)
        nwin = nch // W

        zvec = jnp.zeros((LANES,), jnp.float32)

        @pl.loop(0, CH)
        def _(r):
            @pl.loop(0, d, step=LANES)
            def _(c):
                fbuf[r, pl.ds(c, LANES)] = zvec

        @pl.loop(0, stripe, step=CH)
        def _(r0):
            pltpu.sync_copy(fbuf, acc.at[pl.ds(sid * stripe + r0, CH)])

        plsc.subcore_barrier()

        def load_win(sw, dw, wi, sem):
            off = pl.multiple_of(wi * W, W)
            pltpu.make_async_copy(src_hbm.at[wid, pl.ds(off, W)], sw,
                                  sem).start()
            pltpu.make_async_copy(dst_hbm.at[wid, pl.ds(off, W)], dw,
                                  sem).start()

        def wait_win(sw, dw, sem):
            pltpu.make_async_copy(src_hbm.at[wid, pl.ds(0, W)], sw,
                                  sem).wait()
            pltpu.make_async_copy(dst_hbm.at[wid, pl.ds(0, W)], dw,
                                  sem).wait()

        pltpu.sync_copy(src_hbm.at[wid, pl.ds(0, W)], swin0)
        pltpu.sync_copy(dst_hbm.at[wid, pl.ds(0, W)], dwin0)
        load_win(swin1, dwin1, 1, semB)
        pltpu.make_async_copy(h_hbm.at[swin0.at[0]], buf0, sem0).start()
        pltpu.make_async_copy(h_hbm.at[swin0.at[1]], buf1, sem1).start()

        def process_window(sw, dw, swn, wi):
            for jj in range(0, W, 2):
                for (jo, buf, sem) in ((jj, buf0, sem0), (jj + 1, buf1, sem1)):
                    pltpu.make_async_copy(h_hbm.at[sw.at[jo]], buf, sem).wait()

                    @pl.loop(0, CH)
                    def _(r):
                        for g in range(d // 32):
                            v = buf[r, pl.ds(LANES * g, LANES)]
                            vb = plsc.bitcast(v, jnp.bfloat16)
                            a, b = plsc.unpack(
                                vb, format=plsc.PackFormat.INTERLEAVED)
                            fbuf[r, pl.ds(32 * g, LANES)] = a
                            fbuf[r, pl.ds(32 * g + LANES, LANES)] = b

                    pltpu.sync_copy(fbuf, acc.at[dw.at[jo]], add=True)
                    nj = jo + 2
                    nidx = sw.at[nj] if nj < W else swn.at[nj - W]

                    @pl.when(wi * W + nj < nch)
                    def _():
                        pltpu.make_async_copy(h_hbm.at[nidx], buf, sem).start()

        @pl.loop(0, nwin, step=2)
        def _(w):
            wait_win(swin1, dwin1, semB)
            process_window(swin0, dwin0, swin1, w)

            @pl.when(w + 2 < nwin)
            def _():
                load_win(swin0, dwin0, w + 2, semA)
                wait_win(swin0, dwin0, semA)

            process_window(swin1, dwin1, swin0, w + 1)

            @pl.when(w + 3 < nwin)
            def _():
                load_win(swin1, dwin1, w + 3, semB)

        plsc.subcore_barrier()
        pltpu.sync_copy(acc.at[pl.ds(sid * stripe, stripe)],
                        out_hbm.at[cid, pl.ds(sid * stripe, stripe)])

    return k(h, src_w, dst_w)


def _dot(a, b):
    return jnp.dot(a, b, precision=lax.Precision.HIGHEST,
                   preferred_element_type=jnp.float32)


def _tc_matmul(x, w, br):
    n, d = x.shape

    def body(x_r, w_r, o_r):
        o_r[...] = _dot(x_r[...], w_r[...])

    return pl.pallas_call(
        body,
        grid=(n // br,),
        in_specs=[pl.BlockSpec((br, d), lambda i: (i, 0)),
                  pl.BlockSpec((d, d), lambda i: (0, 0))],
        out_specs=pl.BlockSpec((br, d), lambda i: (i, 0)),
        out_shape=jax.ShapeDtypeStruct((n, d), jnp.float32),
    )(x, w)


def _tc_scale(deg_parts, h, br):
    n, d = h.shape

    def body(d_r, h_r, dis_o, hp_o, hb_o):
        dis = lax.rsqrt(d_r[0] + d_r[1] + 1.0)
        dis_o[...] = dis
        hp = h_r[...] * dis
        hp_o[...] = hp
        hb_o[...] = hp.astype(jnp.bfloat16)

    return pl.pallas_call(
        body,
        grid=(n // br,),
        in_specs=[pl.BlockSpec((2, br, 1), lambda i: (0, i, 0)),
                  pl.BlockSpec((br, d), lambda i: (i, 0))],
        out_specs=[pl.BlockSpec((br, 1), lambda i: (i, 0)),
                   pl.BlockSpec((br, d), lambda i: (i, 0)),
                   pl.BlockSpec((br, d), lambda i: (i, 0))],
        out_shape=[jax.ShapeDtypeStruct((n, 1), jnp.float32),
                   jax.ShapeDtypeStruct((n, d), jnp.float32),
                   jax.ShapeDtypeStruct((n, d), jnp.bfloat16)],
    )(deg_parts, h)


def _tc_finish_mm(parts, hp, dis, b, w, br):
    n, d = hp.shape

    def body(p_r, hp_r, dis_r, b_r, w_r, z_o, hp2_o, hb2_o):
        dis = dis_r[...]
        z = jnp.maximum(dis * (p_r[0] + p_r[1] + hp_r[...]) + b_r[...], 0.0)
        z_o[...] = z
        hp2 = _dot(z, w_r[...]) * dis
        hp2_o[...] = hp2
        hb2_o[...] = hp2.astype(jnp.bfloat16)

    return pl.pallas_call(
        body,
        grid=(n // br,),
        in_specs=[pl.BlockSpec((2, br, d), lambda i: (0, i, 0)),
                  pl.BlockSpec((br, d), lambda i: (i, 0)),
                  pl.BlockSpec((br, 1), lambda i: (i, 0)),
                  pl.BlockSpec((1, d), lambda i: (0, 0)),
                  pl.BlockSpec((d, d), lambda i: (0, 0))],
        out_specs=[pl.BlockSpec((br, d), lambda i: (i, 0)),
                   pl.BlockSpec((br, d), lambda i: (i, 0)),
                   pl.BlockSpec((br, d), lambda i: (i, 0))],
        out_shape=[jax.ShapeDtypeStruct((n, d), jnp.float32),
                   jax.ShapeDtypeStruct((n, d), jnp.float32),
                   jax.ShapeDtypeStruct((n, d), jnp.bfloat16)],
    )(parts, hp, dis, b, w)


def _tc_finish(parts, hp, dis, b, br):
    n, d = hp.shape

    def body(p_r, hp_r, dis_r, b_r, o_r):
        o_r[...] = dis_r[...] * (p_r[0] + p_r[1] + hp_r[...]) + b_r[...]

    return pl.pallas_call(
        body,
        grid=(n // br,),
        in_specs=[pl.BlockSpec((2, br, d), lambda i: (0, i, 0)),
                  pl.BlockSpec((br, d), lambda i: (i, 0)),
                  pl.BlockSpec((br, 1), lambda i: (i, 0)),
                  pl.BlockSpec((1, d), lambda i: (0, 0))],
        out_specs=pl.BlockSpec((br, d), lambda i: (i, 0)),
        out_shape=jax.ShapeDtypeStruct((n, d), jnp.float32),
    )(parts, hp, dis, b)


def kernel(x, edge_index, W1, b1, W2, b2):
    n, d = x.shape
    e = edge_index.shape[1]

    stripe = -(-(n + 1) // (NS * CH)) * CH
    n_acc = NS * stripe
    br = 1000

    src = edge_index[0].astype(jnp.int32)
    dst = edge_index[1].astype(jnp.int32)

    e0_cap = NS * NCH0 * CH
    e1_cap = NS * NCH1 * CH
    e0 = min(e, e0_cap * e // (e0_cap + e1_cap) // CH * CH)

    def side(idx_arr, fill, lo, hi, cap, nch_rows):
        part = idx_arr[lo:hi]
        part = jnp.concatenate(
            [part, jnp.full((cap - (hi - lo),), fill, jnp.int32)])
        part = part.reshape(NS, nch_rows, CH)
        if nch_rows < NCH0:
            part = jnp.concatenate(
                [part, jnp.full((NS, NCH0 - nch_rows, CH), fill, jnp.int32)],
                axis=1)
        return part

    def both_sides(idx_arr, fill):
        a = side(idx_arr, fill, 0, e0, e0_cap, NCH0)
        b = side(idx_arr, fill, e0, e, e1_cap, NCH1)
        return jnp.stack([a, b], axis=1).reshape(NW, NCH0, CH)

    src_w = both_sides(src, 0)
    dst_w = both_sides(dst, n)

    nch_deg = -(-e // (NW * CH))
    e_pad = NW * nch_deg * CH
    dst_deg = jnp.concatenate(
        [dst, jnp.full((e_pad - e,), n, jnp.int32)]).reshape(NW, nch_deg, CH)

    b1r = b1.reshape(1, d).astype(jnp.float32)
    b2r = b2.reshape(1, d).astype(jnp.float32)

    cidx = jnp.arange(d, dtype=jnp.int32)
    pvec = 32 * (cidx // 32) + (cidx % 32) // 2 + 16 * (cidx % 2)

    def pack_bf16(hb):
        return jax.lax.bitcast_convert_type(
            hb.reshape(n, d // 2, 2), jnp.int32)

    deg_parts = _sc_degree(dst_deg, n_acc, nch_deg, stripe)
    h1 = _tc_matmul(x, W1, br)

    dis, h1p, h1b = _tc_scale(deg_parts.reshape(NC, n_acc, 1), h1, br)

    p = _sc_aggregate(pack_bf16(h1b), src_w, dst_w, n_acc, stripe, d)
    z1, h2p, h2b = _tc_finish_mm(p[:, :, pvec], h1p, dis, b1r, W2, br)

    q = _sc_aggregate(pack_bf16(h2b), src_w, dst_w, n_acc, stripe, d)
    out2 = _tc_finish(q[:, :, pvec], h2p, dis, b2r, br)

    return (x, z1, out2)


# speedup vs baseline: 1.3701x; 1.1716x over previous
import functools

import jax
import jax.numpy as jnp
from jax import lax
from jax.experimental import pallas as pl
from jax.experimental.pallas import tpu as pltpu
from jax.experimental.pallas import tpu_sc as plsc

NC = 2
NS = 16
NW = NC * NS
CH = 128
W = 8
LANES = 16
NCH0 = 80
NCH1 = 80


def _sc_degree(dst_w, n_acc, nch, stripe):
    mesh = plsc.VectorSubcoreMesh(core_axis_name="c", subcore_axis_name="s")

    @functools.partial(
        pl.kernel,
        out_type=jax.ShapeDtypeStruct((NC, n_acc), jnp.float32),
        mesh=mesh,
        scratch_types=[
            pltpu.VMEM((nch, CH), jnp.int32),
            pltpu.VMEM((CH,), jnp.float32),
            pltpu.VMEM((stripe,), jnp.float32),
            pltpu.VMEM_SHARED((n_acc,), jnp.float32),
        ],
    )
    def k(dst_hbm, out_hbm, dst_v, ones_v, zero_v, acc):
        cid = lax.axis_index("c")
        sid = lax.axis_index("s")
        wid = sid * NC + cid

        @pl.loop(0, CH, step=LANES)
        def _(c):
            ones_v[pl.ds(c, LANES)] = jnp.ones((LANES,), jnp.float32)

        @pl.loop(0, stripe, step=LANES)
        def _(c):
            zero_v[pl.ds(c, LANES)] = jnp.zeros((LANES,), jnp.float32)

        pltpu.sync_copy(zero_v, acc.at[pl.ds(sid * stripe, stripe)])
        plsc.subcore_barrier()

        pltpu.sync_copy(dst_hbm.at[wid], dst_v)

        @pl.loop(0, nch)
        def _(j):
            pltpu.sync_copy(ones_v, acc.at[dst_v.at[j]], add=True)

        plsc.subcore_barrier()
        pltpu.sync_copy(acc.at[pl.ds(sid * stripe, stripe)],
                        out_hbm.at[cid, pl.ds(sid * stripe, stripe)])

    return k(dst_w)


def _sc_aggregate(h, src_w, dst_w, n_acc, stripe, d):
    mesh = plsc.VectorSubcoreMesh(core_axis_name="c", subcore_axis_name="s")

    @functools.partial(
        pl.kernel,
        out_type=jax.ShapeDtypeStruct((NC, n_acc, d), jnp.float32),
        mesh=mesh,
        scratch_types=[
            pltpu.VMEM((W, CH), jnp.int32),
            pltpu.VMEM((W, CH), jnp.int32),
            pltpu.VMEM((W, CH), jnp.int32),
            pltpu.VMEM((W, CH), jnp.int32),
            pltpu.VMEM((CH, d // 2), jnp.int32),
            pltpu.VMEM((CH, d // 2), jnp.int32),
            pltpu.VMEM((CH, d), jnp.float32),
            pltpu.VMEM_SHARED((n_acc, d), jnp.float32),
            pltpu.SemaphoreType.DMA,
            pltpu.SemaphoreType.DMA,
            pltpu.SemaphoreType.DMA,
            pltpu.SemaphoreType.DMA,
        ],
        compiler_params=pltpu.CompilerParams(use_tc_tiling_on_sc=False,
                                             needs_layout_passes=False),
    )
    def k(h_hbm, src_hbm, dst_hbm, out_hbm, swin0, swin1, dwin0, dwin1,
          buf0, buf1, fbuf, acc, sem0, sem1, semA, semB):
        cid = lax.axis_index("c")
        sid = lax.axis_index("s")
        wid = sid * NC + cid
        nch = jnp.where(cid == 0, NCH0, NCH1)
        nwin = nch // W

        zvec = jnp.zeros((LANES,), jnp.float32)

        @pl.loop(0, CH)
        def _(r):
            @pl.loop(0, d, step=LANES)
            def _(c):
                fbuf[r, pl.ds(c, LANES)] = zvec

        @pl.loop(0, stripe, step=CH)
        def _(r0):
            pltpu.sync_copy(fbuf, acc.at[pl.ds(sid * stripe + r0, CH)])

        plsc.subcore_barrier()

        def load_win(sw, dw, wi, sem):
            off = pl.multiple_of(wi * W, W)
            pltpu.make_async_copy(src_hbm.at[wid, pl.ds(off, W)], sw,
                                  sem).start()
            pltpu.make_async_copy(dst_hbm.at[wid, pl.ds(off, W)], dw,
                                  sem).start()

        def wait_win(sw, dw, sem):
            pltpu.make_async_copy(src_hbm.at[wid, pl.ds(0, W)], sw,
                                  sem).wait()
            pltpu.make_async_copy(dst_hbm.at[wid, pl.ds(0, W)], dw,
                                  sem).wait()

        pltpu.sync_copy(src_hbm.at[wid, pl.ds(0, W)], swin0)
        pltpu.sync_copy(dst_hbm.at[wid, pl.ds(0, W)], dwin0)
        load_win(swin1, dwin1, 1, semB)
        pltpu.make_async_copy(h_hbm.at[swin0.at[0]], buf0, sem0).start()
        pltpu.make_async_copy(h_hbm.at[swin0.at[1]], buf1, sem1).start()

        def process_window(sw, dw, swn, wi):
            for jj in range(0, W, 2):
                for (jo, buf, sem) in ((jj, buf0, sem0), (jj + 1, buf1, sem1)):
                    pltpu.make_async_copy(h_hbm.at[sw.at[jo]], buf, sem).wait()

                    @pl.loop(0, CH)
                    def _(r):
                        for g in range(d // 32):
                            v = buf[r, pl.ds(LANES * g, LANES)]
                            vb = plsc.bitcast(v, jnp.bfloat16)
                            a, b = plsc.unpack(
                                vb, format=plsc.PackFormat.INTERLEAVED)
                            fbuf[r, pl.ds(32 * g, LANES)] = a
                            fbuf[r, pl.ds(32 * g + LANES, LANES)] = b

                    pltpu.sync_copy(fbuf, acc.at[dw.at[jo]], add=True)
                    nj = jo + 2
                    nidx = sw.at[nj] if nj < W else swn.at[nj - W]

                    @pl.when(wi * W + nj < nch)
                    def _():
                        pltpu.make_async_copy(h_hbm.at[nidx], buf, sem).start()

        @pl.loop(0, nwin, step=2)
        def _(w):
            wait_win(swin1, dwin1, semB)
            process_window(swin0, dwin0, swin1, w)

            @pl.when(w + 2 < nwin)
            def _():
                load_win(swin0, dwin0, w + 2, semA)
                wait_win(swin0, dwin0, semA)

            process_window(swin1, dwin1, swin0, w + 1)

            @pl.when(w + 3 < nwin)
            def _():
                load_win(swin1, dwin1, w + 3, semB)

        plsc.subcore_barrier()
        pltpu.sync_copy(acc.at[pl.ds(sid * stripe, stripe)],
                        out_hbm.at[cid, pl.ds(sid * stripe, stripe)])

    return k(h, src_w, dst_w)


def _dot(a, b):
    return jnp.dot(a, b, precision=lax.Precision.HIGHEST,
                   preferred_element_type=jnp.float32)


def _tc_matmul(x, w, br):
    n, d = x.shape

    def body(x_r, w_r, o_r):
        o_r[...] = _dot(x_r[...], w_r[...])

    return pl.pallas_call(
        body,
        grid=(n // br,),
        in_specs=[pl.BlockSpec((br, d), lambda i: (i, 0)),
                  pl.BlockSpec((d, d), lambda i: (0, 0))],
        out_specs=pl.BlockSpec((br, d), lambda i: (i, 0)),
        out_shape=jax.ShapeDtypeStruct((n, d), jnp.float32),
    )(x, w)


def _tc_scale(deg_parts, h, br):
    n, d = h.shape

    def body(d_r, h_r, dis_o, hp_o, hb_o):
        dis = lax.rsqrt(d_r[0] + d_r[1] + 1.0)
        dis_o[...] = dis
        hp = h_r[...] * dis
        hp_o[...] = hp
        hb_o[...] = hp.astype(jnp.bfloat16)

    return pl.pallas_call(
        body,
        grid=(n // br,),
        in_specs=[pl.BlockSpec((2, br, 1), lambda i: (0, i, 0)),
                  pl.BlockSpec((br, d), lambda i: (i, 0))],
        out_specs=[pl.BlockSpec((br, 1), lambda i: (i, 0)),
                   pl.BlockSpec((br, d), lambda i: (i, 0)),
                   pl.BlockSpec((br, d), lambda i: (i, 0))],
        out_shape=[jax.ShapeDtypeStruct((n, 1), jnp.float32),
                   jax.ShapeDtypeStruct((n, d), jnp.float32),
                   jax.ShapeDtypeStruct((n, d), jnp.bfloat16)],
    )(deg_parts, h)


def _tc_finish_mm(parts, hp, dis, b, w, br):
    n, d = hp.shape

    def body(p_r, hp_r, dis_r, b_r, w_r, z_o, hp2_o, hb2_o):
        dis = dis_r[...]
        z = jnp.maximum(dis * (p_r[0] + p_r[1] + hp_r[...]) + b_r[...], 0.0)
        z_o[...] = z
        hp2 = _dot(z, w_r[...]) * dis
        hp2_o[...] = hp2
        hb2_o[...] = hp2.astype(jnp.bfloat16)

    return pl.pallas_call(
        body,
        grid=(n // br,),
        in_specs=[pl.BlockSpec((2, br, d), lambda i: (0, i, 0)),
                  pl.BlockSpec((br, d), lambda i: (i, 0)),
                  pl.BlockSpec((br, 1), lambda i: (i, 0)),
                  pl.BlockSpec((1, d), lambda i: (0, 0)),
                  pl.BlockSpec((d, d), lambda i: (0, 0))],
        out_specs=[pl.BlockSpec((br, d), lambda i: (i, 0)),
                   pl.BlockSpec((br, d), lambda i: (i, 0)),
                   pl.BlockSpec((br, d), lambda i: (i, 0))],
        out_shape=[jax.ShapeDtypeStruct((n, d), jnp.float32),
                   jax.ShapeDtypeStruct((n, d), jnp.float32),
                   jax.ShapeDtypeStruct((n, d), jnp.bfloat16)],
    )(parts, hp, dis, b, w)


def _tc_finish(parts, hp, dis, b, br):
    n, d = hp.shape

    def body(p_r, hp_r, dis_r, b_r, o_r):
        o_r[...] = dis_r[...] * (p_r[0] + p_r[1] + hp_r[...]) + b_r[...]

    return pl.pallas_call(
        body,
        grid=(n // br,),
        in_specs=[pl.BlockSpec((2, br, d), lambda i: (0, i, 0)),
                  pl.BlockSpec((br, d), lambda i: (i, 0)),
                  pl.BlockSpec((br, 1), lambda i: (i, 0)),
                  pl.BlockSpec((1, d), lambda i: (0, 0))],
        out_specs=pl.BlockSpec((br, d), lambda i: (i, 0)),
        out_shape=jax.ShapeDtypeStruct((n, d), jnp.float32),
    )(parts, hp, dis, b)


def kernel(x, edge_index, W1, b1, W2, b2):
    n, d = x.shape
    e = edge_index.shape[1]

    stripe = -(-(n + 1) // (NS * CH)) * CH
    n_acc = NS * stripe
    br = 1000

    src = edge_index[0].astype(jnp.int32)
    dst = edge_index[1].astype(jnp.int32)

    e0_cap = NS * NCH0 * CH
    e1_cap = NS * NCH1 * CH
    e0 = min(e, e0_cap * e // (e0_cap + e1_cap) // CH * CH)

    def side(idx_arr, fill, lo, hi, cap, nch_rows):
        part = idx_arr[lo:hi]
        part = jnp.concatenate(
            [part, jnp.full((cap - (hi - lo),), fill, jnp.int32)])
        part = part.reshape(NS, nch_rows, CH)
        if nch_rows < NCH0:
            part = jnp.concatenate(
                [part, jnp.full((NS, NCH0 - nch_rows, CH), fill, jnp.int32)],
                axis=1)
        return part

    def both_sides(idx_arr, fill):
        a = side(idx_arr, fill, 0, e0, e0_cap, NCH0)
        b = side(idx_arr, fill, e0, e, e1_cap, NCH1)
        return jnp.stack([a, b], axis=1).reshape(NW, NCH0, CH)

    src_w = both_sides(src, 0)
    dst_w = both_sides(dst, n)

    nch_deg = -(-e // (NW * CH))
    e_pad = NW * nch_deg * CH
    dst_deg = jnp.concatenate(
        [dst, jnp.full((e_pad - e,), n, jnp.int32)]).reshape(NW, nch_deg, CH)

    b1r = b1.reshape(1, d).astype(jnp.float32)
    b2r = b2.reshape(1, d).astype(jnp.float32)

    def pack_bf16(hb):
        v = hb.reshape(n, d // 32, 2, 16)
        w = jnp.stack([v[:, :, 0, :], v[:, :, 1, :]], axis=-1)
        return jax.lax.bitcast_convert_type(w, jnp.int32).reshape(n, d // 2)

    deg_parts = _sc_degree(dst_deg, n_acc, nch_deg, stripe)
    h1 = _tc_matmul(x, W1, br)

    dis, h1p, h1b = _tc_scale(deg_parts.reshape(NC, n_acc, 1), h1, br)

    p = _sc_aggregate(pack_bf16(h1b), src_w, dst_w, n_acc, stripe, d)
    z1, h2p, h2b = _tc_finish_mm(p, h1p, dis, b1r, W2, br)

    q = _sc_aggregate(pack_bf16(h2b), src_w, dst_w, n_acc, stripe, d)
    out2 = _tc_finish(q, h2p, dis, b2r, br)

    return (x, z1, out2)


# speedup vs baseline: 1.3714x; 1.0009x over previous
import functools

import jax
import jax.numpy as jnp
from jax import lax
from jax.experimental import pallas as pl
from jax.experimental.pallas import tpu as pltpu
from jax.experimental.pallas import tpu_sc as plsc

NC = 2
NS = 16
NW = NC * NS
CH = 128
W = 8
LANES = 16
NCH0 = 80
NCH1 = 80


def _sc_degree(dst_w, n_acc, nch, stripe):
    mesh = plsc.VectorSubcoreMesh(core_axis_name="c", subcore_axis_name="s")

    @functools.partial(
        pl.kernel,
        out_type=jax.ShapeDtypeStruct((NC, n_acc), jnp.float32),
        mesh=mesh,
        scratch_types=[
            pltpu.VMEM((nch, CH), jnp.int32),
            pltpu.VMEM((CH,), jnp.float32),
            pltpu.VMEM((stripe,), jnp.float32),
            pltpu.VMEM_SHARED((n_acc,), jnp.float32),
        ],
    )
    def k(dst_hbm, out_hbm, dst_v, ones_v, zero_v, acc):
        cid = lax.axis_index("c")
        sid = lax.axis_index("s")
        wid = sid * NC + cid

        @pl.loop(0, CH, step=LANES)
        def _(c):
            ones_v[pl.ds(c, LANES)] = jnp.ones((LANES,), jnp.float32)

        @pl.loop(0, stripe, step=LANES)
        def _(c):
            zero_v[pl.ds(c, LANES)] = jnp.zeros((LANES,), jnp.float32)

        pltpu.sync_copy(zero_v, acc.at[pl.ds(sid * stripe, stripe)])
        plsc.subcore_barrier()

        pltpu.sync_copy(dst_hbm.at[wid], dst_v)

        @pl.loop(0, nch)
        def _(j):
            pltpu.sync_copy(ones_v, acc.at[dst_v.at[j]], add=True)

        plsc.subcore_barrier()
        pltpu.sync_copy(acc.at[pl.ds(sid * stripe, stripe)],
                        out_hbm.at[cid, pl.ds(sid * stripe, stripe)])

    return k(dst_w)


def _sc_aggregate(h, src_w, dst_w, n_acc, stripe, d):
    mesh = plsc.VectorSubcoreMesh(core_axis_name="c", subcore_axis_name="s")

    @functools.partial(
        pl.kernel,
        out_type=jax.ShapeDtypeStruct((NC, n_acc, d), jnp.float32),
        mesh=mesh,
        scratch_types=[
            pltpu.VMEM((W, CH), jnp.int32),
            pltpu.VMEM((W, CH), jnp.int32),
            pltpu.VMEM((W, CH), jnp.int32),
            pltpu.VMEM((W, CH), jnp.int32),
            pltpu.VMEM((CH, d // 2), jnp.int32),
            pltpu.VMEM((CH, d // 2), jnp.int32),
            pltpu.VMEM((CH, d), jnp.float32),
            pltpu.VMEM_SHARED((n_acc, d), jnp.float32),
            pltpu.SemaphoreType.DMA,
            pltpu.SemaphoreType.DMA,
            pltpu.SemaphoreType.DMA,
            pltpu.SemaphoreType.DMA,
        ],
        compiler_params=pltpu.CompilerParams(use_tc_tiling_on_sc=False,
                                             needs_layout_passes=False),
    )
    def k(h_hbm, src_hbm, dst_hbm, out_hbm, swin0, swin1, dwin0, dwin1,
          buf0, buf1, fbuf, acc, sem0, sem1, semA, semB):
        cid = lax.axis_index("c")
        sid = lax.axis_index("s")
        wid = sid * NC + cid
        nch = jnp.where(cid == 0, NCH0, NCH1)
        nwin = nch // W

        zvec = jnp.zeros((LANES,), jnp.float32)

        @pl.loop(0, CH)
        def _(r):
            @pl.loop(0, d, step=LANES)
            def _(c):
                fbuf[r, pl.ds(c, LANES)] = zvec

        @pl.loop(0, stripe, step=CH)
        def _(r0):
            pltpu.sync_copy(fbuf, acc.at[pl.ds(sid * stripe + r0, CH)])

        plsc.subcore_barrier()

        def load_win(sw, dw, wi, sem):
            off = pl.multiple_of(wi * W, W)
            pltpu.make_async_copy(src_hbm.at[wid, pl.ds(off, W)], sw,
                                  sem).start()
            pltpu.make_async_copy(dst_hbm.at[wid, pl.ds(off, W)], dw,
                                  sem).start()

        def wait_win(sw, dw, sem):
            pltpu.make_async_copy(src_hbm.at[wid, pl.ds(0, W)], sw,
                                  sem).wait()
            pltpu.make_async_copy(dst_hbm.at[wid, pl.ds(0, W)], dw,
                                  sem).wait()

        pltpu.sync_copy(src_hbm.at[wid, pl.ds(0, W)], swin0)
        pltpu.sync_copy(dst_hbm.at[wid, pl.ds(0, W)], dwin0)
        load_win(swin1, dwin1, 1, semB)
        pltpu.make_async_copy(h_hbm.at[swin0.at[0]], buf0, sem0).start()
        pltpu.make_async_copy(h_hbm.at[swin0.at[1]], buf1, sem1).start()

        def process_window(sw, dw, swn, wi):
            mask = jnp.full((LANES,), -65536, jnp.int32)
            for jj in range(0, W, 2):
                for (jo, buf, sem) in ((jj, buf0, sem0), (jj + 1, buf1, sem1)):
                    pltpu.make_async_copy(h_hbm.at[sw.at[jo]], buf, sem).wait()

                    @pl.loop(0, CH)
                    def _(r):
                        for g in range(d // 32):
                            v = buf[r, pl.ds(LANES * g, LANES)]
                            lo = plsc.bitcast(v << 16, jnp.float32)
                            hi = plsc.bitcast(v & mask, jnp.float32)
                            fbuf[r, pl.ds(32 * g, LANES)] = lo
                            fbuf[r, pl.ds(32 * g + LANES, LANES)] = hi

                    nj = jo + 2
                    nidx = sw.at[nj] if nj < W else swn.at[nj - W]

                    @pl.when(wi * W + nj < nch)
                    def _():
                        pltpu.make_async_copy(h_hbm.at[nidx], buf, sem).start()

                    pltpu.sync_copy(fbuf, acc.at[dw.at[jo]], add=True)

        @pl.loop(0, nwin, step=2)
        def _(w):
            wait_win(swin1, dwin1, semB)
            process_window(swin0, dwin0, swin1, w)

            @pl.when(w + 2 < nwin)
            def _():
                load_win(swin0, dwin0, w + 2, semA)
                wait_win(swin0, dwin0, semA)

            process_window(swin1, dwin1, swin0, w + 1)

            @pl.when(w + 3 < nwin)
            def _():
                load_win(swin1, dwin1, w + 3, semB)

        plsc.subcore_barrier()
        pltpu.sync_copy(acc.at[pl.ds(sid * stripe, stripe)],
                        out_hbm.at[cid, pl.ds(sid * stripe, stripe)])

    return k(h, src_w, dst_w)


def _dot(a, b):
    return jnp.dot(a, b, precision=lax.Precision.HIGHEST,
                   preferred_element_type=jnp.float32)


def _tc_matmul(x, w, br):
    n, d = x.shape

    def body(x_r, w_r, o_r):
        o_r[...] = _dot(x_r[...], w_r[...])

    return pl.pallas_call(
        body,
        grid=(n // br,),
        in_specs=[pl.BlockSpec((br, d), lambda i: (i, 0)),
                  pl.BlockSpec((d, d), lambda i: (0, 0))],
        out_specs=pl.BlockSpec((br, d), lambda i: (i, 0)),
        out_shape=jax.ShapeDtypeStruct((n, d), jnp.float32),
    )(x, w)


def _tc_scale(deg_parts, h, br):
    n, d = h.shape

    def body(d_r, h_r, dis_o, hp_o, hb_o):
        dis = lax.rsqrt(d_r[0] + d_r[1] + 1.0)
        dis_o[...] = dis
        hp = h_r[...] * dis
        hp_o[...] = hp
        hb_o[...] = hp.astype(jnp.bfloat16)

    return pl.pallas_call(
        body,
        grid=(n // br,),
        in_specs=[pl.BlockSpec((2, br, 1), lambda i: (0, i, 0)),
                  pl.BlockSpec((br, d), lambda i: (i, 0))],
        out_specs=[pl.BlockSpec((br, 1), lambda i: (i, 0)),
                   pl.BlockSpec((br, d), lambda i: (i, 0)),
                   pl.BlockSpec((br, d), lambda i: (i, 0))],
        out_shape=[jax.ShapeDtypeStruct((n, 1), jnp.float32),
                   jax.ShapeDtypeStruct((n, d), jnp.float32),
                   jax.ShapeDtypeStruct((n, d), jnp.bfloat16)],
    )(deg_parts, h)


def _tc_finish_mm(parts, hp, dis, b, w, br):
    n, d = hp.shape

    def body(p_r, hp_r, dis_r, b_r, w_r, z_o, hp2_o, hb2_o):
        dis = dis_r[...]
        z = jnp.maximum(dis * (p_r[0] + p_r[1] + hp_r[...]) + b_r[...], 0.0)
        z_o[...] = z
        hp2 = _dot(z, w_r[...]) * dis
        hp2_o[...] = hp2
        hb2_o[...] = hp2.astype(jnp.bfloat16)

    return pl.pallas_call(
        body,
        grid=(n // br,),
        in_specs=[pl.BlockSpec((2, br, d), lambda i: (0, i, 0)),
                  pl.BlockSpec((br, d), lambda i: (i, 0)),
                  pl.BlockSpec((br, 1), lambda i: (i, 0)),
                  pl.BlockSpec((1, d), lambda i: (0, 0)),
                  pl.BlockSpec((d, d), lambda i: (0, 0))],
        out_specs=[pl.BlockSpec((br, d), lambda i: (i, 0)),
                   pl.BlockSpec((br, d), lambda i: (i, 0)),
                   pl.BlockSpec((br, d), lambda i: (i, 0))],
        out_shape=[jax.ShapeDtypeStruct((n, d), jnp.float32),
                   jax.ShapeDtypeStruct((n, d), jnp.float32),
                   jax.ShapeDtypeStruct((n, d), jnp.bfloat16)],
    )(parts, hp, dis, b, w)


def _tc_finish(parts, hp, dis, b, br):
    n, d = hp.shape

    def body(p_r, hp_r, dis_r, b_r, o_r):
        o_r[...] = dis_r[...] * (p_r[0] + p_r[1] + hp_r[...]) + b_r[...]

    return pl.pallas_call(
        body,
        grid=(n // br,),
        in_specs=[pl.BlockSpec((2, br, d), lambda i: (0, i, 0)),
                  pl.BlockSpec((br, d), lambda i: (i, 0)),
                  pl.BlockSpec((br, 1), lambda i: (i, 0)),
                  pl.BlockSpec((1, d), lambda i: (0, 0))],
        out_specs=pl.BlockSpec((br, d), lambda i: (i, 0)),
        out_shape=jax.ShapeDtypeStruct((n, d), jnp.float32),
    )(parts, hp, dis, b)


def kernel(x, edge_index, W1, b1, W2, b2):
    n, d = x.shape
    e = edge_index.shape[1]

    stripe = -(-(n + 1) // (NS * CH)) * CH
    n_acc = NS * stripe
    br = 1000

    src = edge_index[0].astype(jnp.int32)
    dst = edge_index[1].astype(jnp.int32)

    e0_cap = NS * NCH0 * CH
    e1_cap = NS * NCH1 * CH
    e0 = min(e, e0_cap * e // (e0_cap + e1_cap) // CH * CH)

    def side(idx_arr, fill, lo, hi, cap, nch_rows):
        part = idx_arr[lo:hi]
        part = jnp.concatenate(
            [part, jnp.full((cap - (hi - lo),), fill, jnp.int32)])
        part = part.reshape(NS, nch_rows, CH)
        if nch_rows < NCH0:
            part = jnp.concatenate(
                [part, jnp.full((NS, NCH0 - nch_rows, CH), fill, jnp.int32)],
                axis=1)
        return part

    def both_sides(idx_arr, fill):
        a = side(idx_arr, fill, 0, e0, e0_cap, NCH0)
        b = side(idx_arr, fill, e0, e, e1_cap, NCH1)
        return jnp.stack([a, b], axis=1).reshape(NW, NCH0, CH)

    src_w = both_sides(src, 0)
    dst_w = both_sides(dst, n)

    nch_deg = -(-e // (NW * CH))
    e_pad = NW * nch_deg * CH
    dst_deg = jnp.concatenate(
        [dst, jnp.full((e_pad - e,), n, jnp.int32)]).reshape(NW, nch_deg, CH)

    b1r = b1.reshape(1, d).astype(jnp.float32)
    b2r = b2.reshape(1, d).astype(jnp.float32)

    def pack_bf16(hb):
        v = hb.reshape(n, d // 32, 2, 16)
        w = jnp.stack([v[:, :, 0, :], v[:, :, 1, :]], axis=-1)
        return jax.lax.bitcast_convert_type(w, jnp.int32).reshape(n, d // 2)

    deg_parts = _sc_degree(dst_deg, n_acc, nch_deg, stripe)
    h1 = _tc_matmul(x, W1, br)

    dis, h1p, h1b = _tc_scale(deg_parts.reshape(NC, n_acc, 1), h1, br)

    p = _sc_aggregate(pack_bf16(h1b), src_w, dst_w, n_acc, stripe, d)
    z1, h2p, h2b = _tc_finish_mm(p, h1p, dis, b1r, W2, br)

    q = _sc_aggregate(pack_bf16(h2b), src_w, dst_w, n_acc, stripe, d)
    out2 = _tc_finish(q, h2p, dis, b2r, br)

    return (x, z1, out2)


# speedup vs baseline: 1.3762x; 1.0035x over previous
import functools

import jax
import jax.numpy as jnp
from jax import lax
from jax.experimental import pallas as pl
from jax.experimental.pallas import tpu as pltpu
from jax.experimental.pallas import tpu_sc as plsc

NC = 2
NS = 16
NW = NC * NS
CH = 128
W = 8
LANES = 16
NCH0 = 80
NCH1 = 80


def _sc_degree(dst_w, n_acc, nch, stripe):
    mesh = plsc.VectorSubcoreMesh(core_axis_name="c", subcore_axis_name="s")

    @functools.partial(
        pl.kernel,
        out_type=jax.ShapeDtypeStruct((NC, n_acc), jnp.float32),
        mesh=mesh,
        scratch_types=[
            pltpu.VMEM((nch, CH), jnp.int32),
            pltpu.VMEM((CH,), jnp.float32),
            pltpu.VMEM((stripe,), jnp.float32),
            pltpu.VMEM_SHARED((n_acc,), jnp.float32),
        ],
    )
    def k(dst_hbm, out_hbm, dst_v, ones_v, zero_v, acc):
        cid = lax.axis_index("c")
        sid = lax.axis_index("s")
        wid = sid * NC + cid

        @pl.loop(0, CH, step=LANES)
        def _(c):
            ones_v[pl.ds(c, LANES)] = jnp.ones((LANES,), jnp.float32)

        @pl.loop(0, stripe, step=LANES)
        def _(c):
            zero_v[pl.ds(c, LANES)] = jnp.zeros((LANES,), jnp.float32)

        pltpu.sync_copy(zero_v, acc.at[pl.ds(sid * stripe, stripe)])
        plsc.subcore_barrier()

        pltpu.sync_copy(dst_hbm.at[wid], dst_v)

        @pl.loop(0, nch)
        def _(j):
            pltpu.sync_copy(ones_v, acc.at[dst_v.at[j]], add=True)

        plsc.subcore_barrier()
        pltpu.sync_copy(acc.at[pl.ds(sid * stripe, stripe)],
                        out_hbm.at[cid, pl.ds(sid * stripe, stripe)])

    return k(dst_w)


def _sc_aggregate(h, src_w, dst_w, n_acc, stripe, d):
    mesh = plsc.VectorSubcoreMesh(core_axis_name="c", subcore_axis_name="s")

    @functools.partial(
        pl.kernel,
        out_type=jax.ShapeDtypeStruct((NC, n_acc, d), jnp.float32),
        mesh=mesh,
        scratch_types=[
            pltpu.VMEM((W, CH), jnp.int32),
            pltpu.VMEM((W, CH), jnp.int32),
            pltpu.VMEM((W, CH), jnp.int32),
            pltpu.VMEM((W, CH), jnp.int32),
            pltpu.VMEM((CH, d // 2), jnp.int32),
            pltpu.VMEM((CH, d // 2), jnp.int32),
            pltpu.VMEM((CH, d), jnp.float32),
            pltpu.VMEM_SHARED((n_acc, d), jnp.float32),
            pltpu.SemaphoreType.DMA,
            pltpu.SemaphoreType.DMA,
            pltpu.SemaphoreType.DMA,
            pltpu.SemaphoreType.DMA,
        ],
        compiler_params=pltpu.CompilerParams(use_tc_tiling_on_sc=False,
                                             needs_layout_passes=False),
    )
    def k(h_hbm, src_hbm, dst_hbm, out_hbm, swin0, swin1, dwin0, dwin1,
          buf0, buf1, fbuf, acc, sem0, sem1, semA, semB):
        cid = lax.axis_index("c")
        sid = lax.axis_index("s")
        wid = sid * NC + cid
        nch = jnp.where(cid == 0, NCH0, NCH1)
        nwin = nch // W

        zvec = jnp.zeros((LANES,), jnp.float32)

        @pl.loop(0, CH)
        def _(r):
            @pl.loop(0, d, step=LANES)
            def _(c):
                fbuf[r, pl.ds(c, LANES)] = zvec

        @pl.loop(0, stripe, step=CH)
        def _(r0):
            pltpu.sync_copy(fbuf, acc.at[pl.ds(sid * stripe + r0, CH)])

        plsc.subcore_barrier()

        def load_win(sw, dw, wi, sem):
            off = pl.multiple_of(wi * W, W)
            pltpu.make_async_copy(src_hbm.at[wid, pl.ds(off, W)], sw,
                                  sem).start()
            pltpu.make_async_copy(dst_hbm.at[wid, pl.ds(off, W)], dw,
                                  sem).start()

        def wait_win(sw, dw, sem):
            pltpu.make_async_copy(src_hbm.at[wid, pl.ds(0, W)], sw,
                                  sem).wait()
            pltpu.make_async_copy(dst_hbm.at[wid, pl.ds(0, W)], dw,
                                  sem).wait()

        pltpu.sync_copy(src_hbm.at[wid, pl.ds(0, W)], swin0)
        pltpu.sync_copy(dst_hbm.at[wid, pl.ds(0, W)], dwin0)
        load_win(swin1, dwin1, 1, semB)
        pltpu.make_async_copy(h_hbm.at[swin0.at[0]], buf0, sem0).start()
        pltpu.make_async_copy(h_hbm.at[swin0.at[1]], buf1, sem1).start()

        def process_window(sw, dw, swn, wi):
            mask = jnp.full((LANES,), -65536, jnp.int32)
            for jj in range(0, W, 2):
                for (jo, buf, sem) in ((jj, buf0, sem0), (jj + 1, buf1, sem1)):
                    pltpu.make_async_copy(h_hbm.at[sw.at[jo]], buf, sem).wait()

                    @pl.loop(0, CH)
                    def _(r):
                        for g in range(d // 32):
                            v = buf[r, pl.ds(LANES * g, LANES)]
                            lo = plsc.bitcast(v << 16, jnp.float32)
                            hi = plsc.bitcast(v & mask, jnp.float32)
                            fbuf[r, pl.ds(32 * g, LANES)] = lo
                            fbuf[r, pl.ds(32 * g + LANES, LANES)] = hi

                    nj = jo + 2
                    nidx = sw.at[nj] if nj < W else swn.at[nj - W]

                    @pl.when(wi * W + nj < nch)
                    def _():
                        pltpu.make_async_copy(h_hbm.at[nidx], buf, sem).start()

                    pltpu.sync_copy(fbuf, acc.at[dw.at[jo]], add=True)

        @pl.loop(0, nwin, step=2)
        def _(w):
            wait_win(swin1, dwin1, semB)
            process_window(swin0, dwin0, swin1, w)

            @pl.when(w + 2 < nwin)
            def _():
                load_win(swin0, dwin0, w + 2, semA)
                wait_win(swin0, dwin0, semA)

            process_window(swin1, dwin1, swin0, w + 1)

            @pl.when(w + 3 < nwin)
            def _():
                load_win(swin1, dwin1, w + 3, semB)

        plsc.subcore_barrier()
        pltpu.sync_copy(acc.at[pl.ds(sid * stripe, stripe)],
                        out_hbm.at[cid, pl.ds(sid * stripe, stripe)])

    return k(h, src_w, dst_w)


def _dot(a, b):
    return jnp.dot(a, b, precision=lax.Precision.HIGHEST,
                   preferred_element_type=jnp.float32)


def _tc_matmul(x, w, br):
    n, d = x.shape

    def body(x_r, w_r, o_r):
        o_r[...] = _dot(x_r[...], w_r[...])

    return pl.pallas_call(
        body,
        grid=(n // br,),
        in_specs=[pl.BlockSpec((br, d), lambda i: (i, 0)),
                  pl.BlockSpec((d, d), lambda i: (0, 0))],
        out_specs=pl.BlockSpec((br, d), lambda i: (i, 0)),
        out_shape=jax.ShapeDtypeStruct((n, d), jnp.float32),
    )(x, w)


def _tc_scale(deg_parts, h, br):
    n, d = h.shape

    def body(d_r, h_r, dis_o, hp_o, hb_o):
        dis = lax.rsqrt(d_r[0] + d_r[1] + 1.0)
        dis_o[...] = dis
        hp = h_r[...] * dis
        hp_o[...] = hp
        hb_o[...] = hp.astype(jnp.bfloat16)

    return pl.pallas_call(
        body,
        grid=(n // br,),
        in_specs=[pl.BlockSpec((2, br, 1), lambda i: (0, i, 0)),
                  pl.BlockSpec((br, d), lambda i: (i, 0))],
        out_specs=[pl.BlockSpec((br, 1), lambda i: (i, 0)),
                   pl.BlockSpec((br, d), lambda i: (i, 0)),
                   pl.BlockSpec((br, d), lambda i: (i, 0))],
        out_shape=[jax.ShapeDtypeStruct((n, 1), jnp.float32),
                   jax.ShapeDtypeStruct((n, d), jnp.float32),
                   jax.ShapeDtypeStruct((n, d), jnp.bfloat16)],
    )(deg_parts, h)


def _tc_finish_mm(parts, hp, dis, b, w, br):
    n, d = hp.shape

    def body(p_r, hp_r, dis_r, b_r, w_r, z_o, hp2_o, hb2_o):
        dis = dis_r[...]
        z = jnp.maximum(dis * (p_r[0] + p_r[1] + hp_r[...]) + b_r[...], 0.0)
        z_o[...] = z
        hp2 = _dot(z, w_r[...]) * dis
        hp2_o[...] = hp2
        hb2_o[...] = hp2.astype(jnp.bfloat16)

    return pl.pallas_call(
        body,
        grid=(n // br,),
        in_specs=[pl.BlockSpec((2, br, d), lambda i: (0, i, 0)),
                  pl.BlockSpec((br, d), lambda i: (i, 0)),
                  pl.BlockSpec((br, 1), lambda i: (i, 0)),
                  pl.BlockSpec((1, d), lambda i: (0, 0)),
                  pl.BlockSpec((d, d), lambda i: (0, 0))],
        out_specs=[pl.BlockSpec((br, d), lambda i: (i, 0)),
                   pl.BlockSpec((br, d), lambda i: (i, 0)),
                   pl.BlockSpec((br, d), lambda i: (i, 0))],
        out_shape=[jax.ShapeDtypeStruct((n, d), jnp.float32),
                   jax.ShapeDtypeStruct((n, d), jnp.float32),
                   jax.ShapeDtypeStruct((n, d), jnp.bfloat16)],
    )(parts, hp, dis, b, w)


def _tc_finish(parts, hp, dis, b, br):
    n, d = hp.shape

    def body(p_r, hp_r, dis_r, b_r, o_r):
        o_r[...] = dis_r[...] * (p_r[0] + p_r[1] + hp_r[...]) + b_r[...]

    return pl.pallas_call(
        body,
        grid=(n // br,),
        in_specs=[pl.BlockSpec((2, br, d), lambda i: (0, i, 0)),
                  pl.BlockSpec((br, d), lambda i: (i, 0)),
                  pl.BlockSpec((br, 1), lambda i: (i, 0)),
                  pl.BlockSpec((1, d), lambda i: (0, 0))],
        out_specs=pl.BlockSpec((br, d), lambda i: (i, 0)),
        out_shape=jax.ShapeDtypeStruct((n, d), jnp.float32),
    )(parts, hp, dis, b)


def kernel(x, edge_index, W1, b1, W2, b2):
    n, d = x.shape
    e = edge_index.shape[1]

    stripe = -(-(n + 1) // (NS * CH)) * CH
    n_acc = NS * stripe
    br = 1000

    src = edge_index[0].astype(jnp.int32)
    dst = edge_index[1].astype(jnp.int32)

    e0_cap = NS * NCH0 * CH
    e1_cap = NS * NCH1 * CH
    e0 = min(e, e0_cap * e // (e0_cap + e1_cap) // CH * CH)

    def side(idx_arr, fill, lo, hi, cap, nch_rows):
        part = idx_arr[lo:hi]
        part = jnp.concatenate(
            [part, jnp.full((cap - (hi - lo),), fill, jnp.int32)])
        part = part.reshape(NS, nch_rows, CH)
        if nch_rows < NCH0:
            part = jnp.concatenate(
                [part, jnp.full((NS, NCH0 - nch_rows, CH), fill, jnp.int32)],
                axis=1)
        return part

    def both_sides(idx_arr, fill):
        a = side(idx_arr, fill, 0, e0, e0_cap, NCH0)
        b = side(idx_arr, fill, e0, e, e1_cap, NCH1)
        return jnp.stack([a, b], axis=1).reshape(NW, NCH0, CH)

    src_w = both_sides(src, 0)
    dst_w = both_sides(dst, n)

    b1r = b1.reshape(1, d).astype(jnp.float32)
    b2r = b2.reshape(1, d).astype(jnp.float32)

    def pack_bf16(hb):
        v = hb.reshape(n, d // 32, 2, 16)
        w = jnp.stack([v[:, :, 0, :], v[:, :, 1, :]], axis=-1)
        return jax.lax.bitcast_convert_type(w, jnp.int32).reshape(n, d // 2)

    deg_parts = _sc_degree(dst_w, n_acc, NCH0, stripe)
    h1 = _tc_matmul(x, W1, br)

    dis, h1p, h1b = _tc_scale(deg_parts.reshape(NC, n_acc, 1), h1, br)

    p = _sc_aggregate(pack_bf16(h1b), src_w, dst_w, n_acc, stripe, d)
    z1, h2p, h2b = _tc_finish_mm(p, h1p, dis, b1r, W2, br)

    q = _sc_aggregate(pack_bf16(h2b), src_w, dst_w, n_acc, stripe, d)
    out2 = _tc_finish(q, h2p, dis, b2r, br)

    return (x, z1, out2)


# speedup vs baseline: 1.3929x; 1.0121x over previous
import functools

import jax
import jax.numpy as jnp
from jax import lax
from jax.experimental import pallas as pl
from jax.experimental.pallas import tpu as pltpu
from jax.experimental.pallas import tpu_sc as plsc

NC = 2
NS = 16
NW = NC * NS
CH = 128
W = 8
LANES = 16
NCH0 = 80
NCH1 = 80


def _sc_degree(dst_w, n_acc, nch, stripe):
    mesh = plsc.VectorSubcoreMesh(core_axis_name="c", subcore_axis_name="s")

    @functools.partial(
        pl.kernel,
        out_type=jax.ShapeDtypeStruct((NC, n_acc), jnp.float32),
        mesh=mesh,
        scratch_types=[
            pltpu.VMEM((nch, CH), jnp.int32),
            pltpu.VMEM((CH,), jnp.float32),
            pltpu.VMEM((stripe,), jnp.float32),
            pltpu.VMEM_SHARED((n_acc,), jnp.float32),
        ],
    )
    def k(dst_hbm, out_hbm, dst_v, ones_v, zero_v, acc):
        cid = lax.axis_index("c")
        sid = lax.axis_index("s")
        wid = sid * NC + cid

        @pl.loop(0, CH, step=LANES)
        def _(c):
            ones_v[pl.ds(c, LANES)] = jnp.ones((LANES,), jnp.float32)

        @pl.loop(0, stripe, step=LANES)
        def _(c):
            zero_v[pl.ds(c, LANES)] = jnp.zeros((LANES,), jnp.float32)

        pltpu.sync_copy(zero_v, acc.at[pl.ds(sid * stripe, stripe)])
        plsc.subcore_barrier()

        pltpu.sync_copy(dst_hbm.at[wid], dst_v)

        @pl.loop(0, nch)
        def _(j):
            pltpu.sync_copy(ones_v, acc.at[dst_v.at[j]], add=True)

        plsc.subcore_barrier()
        pltpu.sync_copy(acc.at[pl.ds(sid * stripe, stripe)],
                        out_hbm.at[cid, pl.ds(sid * stripe, stripe)])

    return k(dst_w)


def _sc_aggregate(h, src_w, dst_w, n_acc, stripe, d):
    mesh = plsc.VectorSubcoreMesh(core_axis_name="c", subcore_axis_name="s")

    @functools.partial(
        pl.kernel,
        out_type=jax.ShapeDtypeStruct((NC, n_acc, d), jnp.float32),
        mesh=mesh,
        scratch_types=[
            pltpu.VMEM((W, CH), jnp.int32),
            pltpu.VMEM((W, CH), jnp.int32),
            pltpu.VMEM((W, CH), jnp.int32),
            pltpu.VMEM((W, CH), jnp.int32),
            pltpu.VMEM((CH, d // 2), jnp.int32),
            pltpu.VMEM((CH, d // 2), jnp.int32),
            pltpu.VMEM((CH, d), jnp.float32),
            pltpu.VMEM_SHARED((n_acc, d), jnp.float32),
            pltpu.SemaphoreType.DMA,
            pltpu.SemaphoreType.DMA,
            pltpu.SemaphoreType.DMA,
            pltpu.SemaphoreType.DMA,
        ],
        compiler_params=pltpu.CompilerParams(use_tc_tiling_on_sc=False,
                                             needs_layout_passes=False),
    )
    def k(h_hbm, src_hbm, dst_hbm, out_hbm, swin0, swin1, dwin0, dwin1,
          buf0, buf1, fbuf, acc, sem0, sem1, semA, semB):
        cid = lax.axis_index("c")
        sid = lax.axis_index("s")
        wid = sid * NC + cid
        nch = jnp.where(cid == 0, NCH0, NCH1)
        nwin = nch // W

        zvec = jnp.zeros((LANES,), jnp.float32)

        @pl.loop(0, CH)
        def _(r):
            @pl.loop(0, d, step=LANES)
            def _(c):
                fbuf[r, pl.ds(c, LANES)] = zvec

        @pl.loop(0, stripe, step=CH)
        def _(r0):
            pltpu.sync_copy(fbuf, acc.at[pl.ds(sid * stripe + r0, CH)])

        plsc.subcore_barrier()

        def load_win(sw, dw, wi, sem):
            off = pl.multiple_of(wi * W, W)
            pltpu.make_async_copy(src_hbm.at[wid, pl.ds(off, W)], sw,
                                  sem).start()
            pltpu.make_async_copy(dst_hbm.at[wid, pl.ds(off, W)], dw,
                                  sem).start()

        def wait_win(sw, dw, sem):
            pltpu.make_async_copy(src_hbm.at[wid, pl.ds(0, W)], sw,
                                  sem).wait()
            pltpu.make_async_copy(dst_hbm.at[wid, pl.ds(0, W)], dw,
                                  sem).wait()

        pltpu.sync_copy(src_hbm.at[wid, pl.ds(0, W)], swin0)
        pltpu.sync_copy(dst_hbm.at[wid, pl.ds(0, W)], dwin0)
        load_win(swin1, dwin1, 1, semB)
        pltpu.make_async_copy(h_hbm.at[swin0.at[0]], buf0, sem0).start()
        pltpu.make_async_copy(h_hbm.at[swin0.at[1]], buf1, sem1).start()

        def process_window(sw, dw, swn, wi):
            mask = jnp.full((LANES,), -65536, jnp.int32)
            for jj in range(0, W, 2):
                for (jo, buf, sem) in ((jj, buf0, sem0), (jj + 1, buf1, sem1)):
                    pltpu.make_async_copy(h_hbm.at[sw.at[jo]], buf, sem).wait()

                    @pl.loop(0, CH)
                    def _(r):
                        for g in range(d // 32):
                            v = buf[r, pl.ds(LANES * g, LANES)]
                            lo = plsc.bitcast(v << 16, jnp.float32)
                            hi = plsc.bitcast(v & mask, jnp.float32)
                            fbuf[r, pl.ds(32 * g, LANES)] = lo
                            fbuf[r, pl.ds(32 * g + LANES, LANES)] = hi

                    nj = jo + 2
                    nidx = sw.at[nj] if nj < W else swn.at[nj - W]

                    @pl.when(wi * W + nj < nch)
                    def _():
                        pltpu.make_async_copy(h_hbm.at[nidx], buf, sem).start()

                    pltpu.sync_copy(fbuf, acc.at[dw.at[jo]], add=True)

        @pl.loop(0, nwin, step=2)
        def _(w):
            wait_win(swin1, dwin1, semB)
            process_window(swin0, dwin0, swin1, w)

            @pl.when(w + 2 < nwin)
            def _():
                load_win(swin0, dwin0, w + 2, semA)
                wait_win(swin0, dwin0, semA)

            process_window(swin1, dwin1, swin0, w + 1)

            @pl.when(w + 3 < nwin)
            def _():
                load_win(swin1, dwin1, w + 3, semB)

        plsc.subcore_barrier()
        pltpu.sync_copy(acc.at[pl.ds(sid * stripe, stripe)],
                        out_hbm.at[cid, pl.ds(sid * stripe, stripe)])

    return k(h, src_w, dst_w)


def _dot(a, b):
    return jnp.dot(a, b, precision=lax.Precision.HIGHEST,
                   preferred_element_type=jnp.float32)


def _tc_matmul(x, w, br):
    n, d = x.shape

    def body(x_r, w_r, o_r):
        o_r[...] = _dot(x_r[...], w_r[...])

    return pl.pallas_call(
        body,
        grid=(n // br,),
        in_specs=[pl.BlockSpec((br, d), lambda i: (i, 0)),
                  pl.BlockSpec((d, d), lambda i: (0, 0))],
        out_specs=pl.BlockSpec((br, d), lambda i: (i, 0)),
        out_shape=jax.ShapeDtypeStruct((n, d), jnp.float32),
    )(x, w)


def _tc_scale(deg_parts, h, br):
    n, d = h.shape

    def body(d_r, h_r, dis_o, hp_o, hb_o):
        dis = lax.rsqrt(d_r[0] + d_r[1] + 1.0)
        dis_o[...] = dis
        hp = h_r[...] * dis
        hp_o[...] = hp
        hb_o[...] = hp.astype(jnp.bfloat16)

    return pl.pallas_call(
        body,
        grid=(n // br,),
        in_specs=[pl.BlockSpec((2, br, 1), lambda i: (0, i, 0)),
                  pl.BlockSpec((br, d), lambda i: (i, 0))],
        out_specs=[pl.BlockSpec((br, 1), lambda i: (i, 0)),
                   pl.BlockSpec((br, d), lambda i: (i, 0)),
                   pl.BlockSpec((br, d), lambda i: (i, 0))],
        out_shape=[jax.ShapeDtypeStruct((n, 1), jnp.float32),
                   jax.ShapeDtypeStruct((n, d), jnp.float32),
                   jax.ShapeDtypeStruct((n, d), jnp.bfloat16)],
    )(deg_parts, h)


def _tc_finish_mm(parts, hp, dis, b, w, br):
    n, d = hp.shape

    def body(p_r, hp_r, dis_r, b_r, w_r, z_o, hp2_o, hb2_o):
        dis = dis_r[...]
        z = jnp.maximum(dis * (p_r[0] + p_r[1] + hp_r[...]) + b_r[...], 0.0)
        z_o[...] = z
        hp2 = _dot(z, w_r[...]) * dis
        hp2_o[...] = hp2
        hb2_o[...] = hp2.astype(jnp.bfloat16)

    return pl.pallas_call(
        body,
        grid=(n // br,),
        in_specs=[pl.BlockSpec((2, br, d), lambda i: (0, i, 0)),
                  pl.BlockSpec((br, d), lambda i: (i, 0)),
                  pl.BlockSpec((br, 1), lambda i: (i, 0)),
                  pl.BlockSpec((1, d), lambda i: (0, 0)),
                  pl.BlockSpec((d, d), lambda i: (0, 0))],
        out_specs=[pl.BlockSpec((br, d), lambda i: (i, 0)),
                   pl.BlockSpec((br, d), lambda i: (i, 0)),
                   pl.BlockSpec((br, d), lambda i: (i, 0))],
        out_shape=[jax.ShapeDtypeStruct((n, d), jnp.float32),
                   jax.ShapeDtypeStruct((n, d), jnp.float32),
                   jax.ShapeDtypeStruct((n, d), jnp.bfloat16)],
    )(parts, hp, dis, b, w)


def _tc_finish(parts, hp, dis, b, br):
    n, d = hp.shape

    def body(p_r, hp_r, dis_r, b_r, o_r):
        o_r[...] = dis_r[...] * (p_r[0] + p_r[1] + hp_r[...]) + b_r[...]

    return pl.pallas_call(
        body,
        grid=(n // br,),
        in_specs=[pl.BlockSpec((2, br, d), lambda i: (0, i, 0)),
                  pl.BlockSpec((br, d), lambda i: (i, 0)),
                  pl.BlockSpec((br, 1), lambda i: (i, 0)),
                  pl.BlockSpec((1, d), lambda i: (0, 0))],
        out_specs=pl.BlockSpec((br, d), lambda i: (i, 0)),
        out_shape=jax.ShapeDtypeStruct((n, d), jnp.float32),
    )(parts, hp, dis, b)


def kernel(x, edge_index, W1, b1, W2, b2):
    n, d = x.shape
    e = edge_index.shape[1]

    stripe = -(-(n + 1) // (NS * CH)) * CH
    n_acc = NS * stripe
    br = 2000

    src = edge_index[0].astype(jnp.int32)
    dst = edge_index[1].astype(jnp.int32)

    e0_cap = NS * NCH0 * CH
    e1_cap = NS * NCH1 * CH
    e0 = min(e, e0_cap * e // (e0_cap + e1_cap) // CH * CH)

    def side(idx_arr, fill, lo, hi, cap, nch_rows):
        part = idx_arr[lo:hi]
        part = jnp.concatenate(
            [part, jnp.full((cap - (hi - lo),), fill, jnp.int32)])
        part = part.reshape(NS, nch_rows, CH)
        if nch_rows < NCH0:
            part = jnp.concatenate(
                [part, jnp.full((NS, NCH0 - nch_rows, CH), fill, jnp.int32)],
                axis=1)
        return part

    def both_sides(idx_arr, fill):
        a = side(idx_arr, fill, 0, e0, e0_cap, NCH0)
        b = side(idx_arr, fill, e0, e, e1_cap, NCH1)
        return jnp.stack([a, b], axis=1).reshape(NW, NCH0, CH)

    src_w = both_sides(src, 0)
    dst_w = both_sides(dst, n)

    b1r = b1.reshape(1, d).astype(jnp.float32)
    b2r = b2.reshape(1, d).astype(jnp.float32)

    def pack_bf16(hb):
        v = hb.reshape(n, d // 32, 2, 16)
        w = jnp.stack([v[:, :, 0, :], v[:, :, 1, :]], axis=-1)
        return jax.lax.bitcast_convert_type(w, jnp.int32).reshape(n, d // 2)

    deg_parts = _sc_degree(dst_w, n_acc, NCH0, stripe)
    h1 = _tc_matmul(x, W1, br)

    dis, h1p, h1b = _tc_scale(deg_parts.reshape(NC, n_acc, 1), h1, br)

    p = _sc_aggregate(pack_bf16(h1b), src_w, dst_w, n_acc, stripe, d)
    z1, h2p, h2b = _tc_finish_mm(p, h1p, dis, b1r, W2, br)

    q = _sc_aggregate(pack_bf16(h2b), src_w, dst_w, n_acc, stripe, d)
    out2 = _tc_finish(q, h2p, dis, b2r, br)

    return (x, z1, out2)
